# baseline (speedup 1.0000x reference)
import functools

import jax
import jax.numpy as jnp
from jax.experimental import pallas as pl
from jax.experimental.pallas import tpu as pltpu

EPS = 1e-8
W_N = 255.0
W_HALF = 128.0


def _fused_kernel(scal_ref, w_ref, bias_ref, x_ref, out_ref, qw_ref, abc_ref,
                  *, in_f, n_prep):
    i = pl.program_id(0)
    s_x = scal_ref[0]
    z_x = scal_ref[1]

    @pl.when(i < n_prep)
    def _prep():
        w = w_ref[...]
        bw = w.shape[0]
        wmin = jnp.min(w, axis=1, keepdims=True)
        wmax = jnp.max(w, axis=1, keepdims=True)
        s_w = W_N / jnp.maximum(wmax - wmin, EPS)
        z_w = jnp.round(s_w * wmin) + W_HALF
        qw = jnp.clip(jnp.round(s_w * w - z_w), -W_HALF, W_HALF - 1.0)
        qw_ref[:, pl.ds(i * bw, bw)] = (qw + z_w).T.astype(jnp.bfloat16)
        qs = jnp.sum(qw, axis=1, keepdims=True)
        a = 1.0 / (s_x * s_w)
        c = (z_x * qs + (z_x * float(in_f)) * z_w) * a
        bcol = bias_ref[0:1, pl.ds(i * bw, bw)]
        abc_ref[:, pl.ds(i * bw, bw)] = jnp.concatenate(
            [a.T, c.T + bcol, jnp.zeros((6, bw), jnp.float32)], axis=0)

    @pl.when(i >= n_prep)
    def _matmul():
        lo = scal_ref[2]
        hi = scal_ref[3]
        x = x_ref[...]
        qx = jnp.clip(jnp.round(s_x * x - z_x), lo, hi)
        acc = jax.lax.dot_general(
            qx.astype(jnp.bfloat16), qw_ref[...],
            dimension_numbers=(((1,), (0,)), ((), ())),
            preferred_element_type=jnp.float32)
        a = abc_ref[0:1, :]
        c = abc_ref[1:2, :]
        out_ref[...] = acc * a + c


def kernel(x, weight, bias, x_min, x_max, activation_bit):
    tok, in_f = x.shape
    out_f = weight.shape[0]

    ka = jnp.asarray(activation_bit, jnp.float32)
    n_a = jnp.exp2(ka) - 1.0
    half_a = jnp.exp2(ka - 1.0)
    s_x = n_a / jnp.maximum(x_max[0] - x_min[0], EPS)
    z_x = jnp.round(s_x * x_min[0]) + half_a
    scal = jnp.stack([s_x, z_x, -half_a, half_a - 1.0]).astype(jnp.float32)
    bias2 = bias.reshape(1, out_f)

    bw = min(256, out_f)
    bm = min(256, tok)
    n_prep = out_f // bw
    n_mm = tok // bm

    out = pl.pallas_call(
        functools.partial(_fused_kernel, in_f=in_f, n_prep=n_prep),
        grid=(n_prep + n_mm,),
        in_specs=[
            pl.BlockSpec(memory_space=pltpu.SMEM),
            pl.BlockSpec((bw, in_f), lambda i: (jnp.minimum(i, n_prep - 1), 0)),
            pl.BlockSpec((1, out_f), lambda i: (0, 0)),
            pl.BlockSpec((bm, in_f), lambda i: (jnp.maximum(i - n_prep, 0), 0)),
        ],
        out_specs=pl.BlockSpec((bm, out_f),
                               lambda i: (jnp.maximum(i - n_prep, 0), 0),
                               pipeline_mode=pl.Buffered(buffer_count=1)),
        out_shape=jax.ShapeDtypeStruct((tok, out_f), jnp.float32),
        scratch_shapes=[
            pltpu.VMEM((in_f, out_f), jnp.bfloat16),
            pltpu.VMEM((8, out_f), jnp.float32),
        ],
        compiler_params=pltpu.CompilerParams(
            dimension_semantics=("arbitrary",)),
    )(scal, weight, bias2, x)
    return out


# speedup vs baseline: 1.1631x; 1.1631x over previous
import functools

import jax
import jax.numpy as jnp
from jax.experimental import pallas as pl
from jax.experimental.pallas import tpu as pltpu

EPS = 1e-8
W_N = 255.0
W_HALF = 128.0


def _fused_kernel(scal_ref, w_ref, bias_ref, x_ref, out_ref, qw_ref, abc_ref,
                  *, in_f, n_prep):
    i = pl.program_id(0)
    s_x = scal_ref[0]
    z_x = scal_ref[1]

    @pl.when(i < n_prep)
    def _prep():
        w = w_ref[...]
        bw = w.shape[0]
        wmin = jnp.min(w, axis=1, keepdims=True)
        wmax = jnp.max(w, axis=1, keepdims=True)
        s_w = W_N / jnp.maximum(wmax - wmin, EPS)
        z_w = jnp.round(s_w * wmin) + W_HALF
        qw = jnp.clip(jnp.round(s_w * w - z_w), -W_HALF, W_HALF - 1.0)
        qw_ref[:, pl.ds(i * bw, bw)] = (qw + z_w).T.astype(jnp.bfloat16)
        qs = jnp.sum(qw, axis=1, keepdims=True)
        a = 1.0 / (s_x * s_w)
        c = (z_x * qs + (z_x * float(in_f)) * z_w) * a
        bcol = bias_ref[0:1, pl.ds(i * bw, bw)]
        abc_ref[:, pl.ds(i * bw, bw)] = jnp.concatenate(
            [a.T, c.T + bcol, jnp.zeros((6, bw), jnp.float32)], axis=0)

    @pl.when(i >= n_prep)
    def _matmul():
        lo = scal_ref[2]
        hi = scal_ref[3]
        x = x_ref[...]
        qx = jnp.clip(jnp.round(s_x * x - z_x), lo, hi)
        acc = jax.lax.dot_general(
            qx.astype(jnp.bfloat16), qw_ref[...],
            dimension_numbers=(((1,), (0,)), ((), ())),
            preferred_element_type=jnp.float32)
        a = abc_ref[0:1, :]
        c = abc_ref[1:2, :]
        out_ref[...] = acc * a + c


def kernel(x, weight, bias, x_min, x_max, activation_bit):
    tok, in_f = x.shape
    out_f = weight.shape[0]

    ka = jnp.asarray(activation_bit, jnp.float32)
    n_a = jnp.exp2(ka) - 1.0
    half_a = jnp.exp2(ka - 1.0)
    s_x = n_a / jnp.maximum(x_max[0] - x_min[0], EPS)
    z_x = jnp.round(s_x * x_min[0]) + half_a
    scal = jnp.stack([s_x, z_x, -half_a, half_a - 1.0]).astype(jnp.float32)
    bias2 = bias.reshape(1, out_f)

    bw = min(128, out_f)
    bm = min(256, tok)
    n_prep = out_f // bw
    n_mm = tok // bm

    out = pl.pallas_call(
        functools.partial(_fused_kernel, in_f=in_f, n_prep=n_prep),
        grid=(n_prep + n_mm,),
        in_specs=[
            pl.BlockSpec(memory_space=pltpu.SMEM),
            pl.BlockSpec((bw, in_f), lambda i: (jnp.minimum(i, n_prep - 1), 0)),
            pl.BlockSpec((1, out_f), lambda i: (0, 0)),
            pl.BlockSpec((bm, in_f), lambda i: (jnp.maximum(i - n_prep, 0), 0)),
        ],
        out_specs=pl.BlockSpec((bm, out_f),
                               lambda i: (jnp.maximum(i - n_prep, 0), 0)),
        out_shape=jax.ShapeDtypeStruct((tok, out_f), jnp.float32),
        scratch_shapes=[
            pltpu.VMEM((in_f, out_f), jnp.bfloat16),
            pltpu.VMEM((8, out_f), jnp.float32),
        ],
        compiler_params=pltpu.CompilerParams(
            dimension_semantics=("arbitrary",)),
    )(scal, weight, bias2, x)
    return out
